# baseline (speedup 1.0000x reference)
import functools

import jax
import jax.numpy as jnp
from jax import lax
from jax.experimental import pallas as pl
from jax.experimental.pallas import tpu as pltpu
from jax.experimental.pallas import tpu_sc as plsc

BATCH = 16384
DIM = 64
LANES = 16
GRPS = DIM // LANES

_info = plsc.get_sparse_core_info()
NC, NS = _info.num_cores, _info.num_subcores
NW = NC * NS
PER_W = BATCH // NW
CHUNK = 256
NCHUNK = PER_W // CHUNK
KIDX = 128
NK = CHUNK // KIDX


def _tec_body(head_hbm, rel_hbm, tail_hbm, ent_hbm, rel_emb_hbm,
              rel_hyper_hbm, out_hbm,
              hidx, tidx, ridx, h_v, t_v, r_v, rh_v, sem):
    wid = lax.axis_index("s") * NC + lax.axis_index("c")
    wbase = wid * PER_W
    lane = lax.iota(jnp.int32, LANES)
    perms = [jnp.bitwise_xor(lane, s) for s in (1, 2, 4, 8)]

    for c in range(NCHUNK):
        base = wbase + c * CHUNK
        for k in range(NK):
            off = base + k * KIDX
            pltpu.sync_copy(head_hbm.at[pl.ds(off, KIDX)], hidx.at[k])
            pltpu.sync_copy(tail_hbm.at[pl.ds(off, KIDX)], tidx.at[k])
            pltpu.sync_copy(rel_hbm.at[pl.ds(off, KIDX)], ridx.at[k])
        cps = []
        for k in range(NK):
            dst = pl.ds(k * KIDX, KIDX)
            cps.append(pltpu.async_copy(ent_hbm.at[hidx.at[k]],
                                        h_v.at[dst], sem))
            cps.append(pltpu.async_copy(ent_hbm.at[tidx.at[k]],
                                        t_v.at[dst], sem))
            cps.append(pltpu.async_copy(rel_emb_hbm.at[ridx.at[k]],
                                        r_v.at[dst], sem))
            cps.append(pltpu.async_copy(rel_hyper_hbm.at[ridx.at[k]],
                                        rh_v.at[dst], sem))
        for cp in cps:
            cp.wait()

        def row(i, _):
            u = []
            rh = []
            for j in range(GRPS):
                sl = pl.ds(j * LANES, LANES)
                u.append(h_v[i, sl] - t_v[i, sl])
                rh.append(rh_v[i, sl])
            acc = u[0] * rh[0]
            for j in range(1, GRPS):
                acc = acc + u[j] * rh[j]
            for p in perms:
                acc = acc + acc.at[p].get(mode="promise_in_bounds")
            for j in range(GRPS):
                sl = pl.ds(j * LANES, LANES)
                h_v[i, sl] = u[j] + r_v[i, sl] - rh[j] * acc
            return 0

        lax.fori_loop(0, CHUNK, row, 0)
        pltpu.sync_copy(h_v, out_hbm.at[pl.ds(base, CHUNK)])


@jax.jit
def kernel(head, relation, tail, ent_emb, rel_emb, rel_hyper):
    mesh = plsc.VectorSubcoreMesh(core_axis_name="c", subcore_axis_name="s")
    run = functools.partial(
        pl.kernel,
        mesh=mesh,
        out_type=jax.ShapeDtypeStruct((BATCH, DIM), jnp.float32),
        scratch_types=[
            pltpu.VMEM((NK, KIDX), jnp.int32),
            pltpu.VMEM((NK, KIDX), jnp.int32),
            pltpu.VMEM((NK, KIDX), jnp.int32),
            pltpu.VMEM((CHUNK, DIM), jnp.float32),
            pltpu.VMEM((CHUNK, DIM), jnp.float32),
            pltpu.VMEM((CHUNK, DIM), jnp.float32),
            pltpu.VMEM((CHUNK, DIM), jnp.float32),
            pltpu.SemaphoreType.DMA,
        ],
        compiler_params=pltpu.CompilerParams(use_tc_tiling_on_sc=False),
    )(_tec_body)
    return run(head.astype(jnp.int32), relation.astype(jnp.int32),
               tail.astype(jnp.int32), ent_emb, rel_emb, rel_hyper)
